# baseline (speedup 1.0000x reference)
import functools

import jax
import jax.numpy as jnp
from jax import lax
from jax.experimental import pallas as pl
from jax.experimental.pallas import tpu as pltpu
from jax.experimental.pallas import tpu_sc as plsc

D = 32
NW = 32
CHUNK_ROWS = 32
NBUF = 3
LANES = 16
W1 = 512


def _make_transpose(V):
    n_full = V // W1
    n_iter = -(-n_full // NW)
    n_pair = -(-n_iter // 2)
    mesh = plsc.VectorSubcoreMesh(core_axis_name="c", subcore_axis_name="s")

    @functools.partial(
        pl.kernel,
        mesh=mesh,
        out_type=jax.ShapeDtypeStruct((V * D,), jnp.float32),
        scratch_types=[pltpu.VMEM((8, W1), jnp.float32)] * 8
        + [pltpu.VMEM((W1 * D,), jnp.float32)] * 2
        + [pltpu.SemaphoreType.DMA] * 4,
        compiler_params=pltpu.CompilerParams(needs_layout_passes=False),
    )
    def k(tt_hbm, tl_hbm, out_hbm, *scr):
        tin = [scr[0:4], scr[4:8]]
        tout = scr[8:10]
        isem = scr[10:12]
        osem = scr[12:14]
        wid = lax.axis_index("s") * 2 + lax.axis_index("c")
        lane32 = lax.iota(jnp.int32, LANES) * D

        def blk_of(kk):
            return wid + NW * kk

        def start_in(blk, s):
            c0 = blk * W1
            for db in range(4):
                pltpu.async_copy(tt_hbm.at[pl.ds(8 * db, 8), pl.ds(c0, W1)],
                                 tin[s][db], isem[s])

        def wait_in(s):
            for db in range(4):
                pltpu.make_async_copy(
                    tt_hbm.at[pl.ds(8 * db, 8), pl.ds(0, W1)],
                    tin[s][db], isem[s]).wait()

        def wait_out(s):
            pltpu.make_async_copy(
                tout[s], out_hbm.at[pl.ds(0, W1 * D)], osem[s]).wait()

        def transpose_into(tin_set, tout_b):
            def body(g, carry):
                base = g * (LANES * D)
                for db in range(4):
                    for e in range(8):
                        v = tin_set[db][e, pl.ds(g * LANES, LANES)]
                        plsc.store_scatter(
                            tout_b, [lane32 + (base + 8 * db + e)], v)
                return carry

            lax.fori_loop(0, W1 // LANES, body, 0)

        @pl.when(blk_of(0) < n_full)
        def _():
            start_in(blk_of(0), 0)

        def pair(t, carry):
            for par in range(2):
                kk = 2 * t + par
                nkk = kk + 1

                @pl.when(blk_of(nkk) < n_full)
                def _():
                    start_in(blk_of(nkk), 1 - par)

                @pl.when(blk_of(kk) < n_full)
                def _():
                    wait_in(par)

                    @pl.when(t >= 1)
                    def _():
                        wait_out(par)

                    transpose_into(tin[par], tout[par])
                    pltpu.async_copy(
                        tout[par],
                        out_hbm.at[pl.ds(blk_of(kk) * (W1 * D), W1 * D)],
                        osem[par])
            return carry

        lax.fori_loop(0, n_pair, pair, 0)

        @pl.when((wid == NW - 1) & (V % W1 != 0))
        def _():
            c0 = V - 128
            wait_out(0)
            for db in range(4):
                pltpu.async_copy(tl_hbm.at[pl.ds(8 * db, 8)],
                                 tin[0][db].at[:, pl.ds(0, 128)], isem[0])
            for db in range(4):
                pltpu.make_async_copy(
                    tl_hbm.at[pl.ds(8 * db, 8)],
                    tin[0][db].at[:, pl.ds(0, 128)], isem[0]).wait()

            def tb(g, carry):
                base = g * (LANES * D)
                for db in range(4):
                    for e in range(8):
                        v = tin[0][db][e, pl.ds(g * LANES, LANES)]
                        plsc.store_scatter(
                            tout[0], [lane32 + (base + 8 * db + e)], v)
                return carry

            lax.fori_loop(0, 128 // LANES, tb, 0)
            pltpu.async_copy(tout[0].at[pl.ds(0, 128 * D)],
                             out_hbm.at[pl.ds(c0 * D, 128 * D)], osem[0])

        tail_w = (wid == NW - 1) & (V % W1 != 0)

        @pl.when(tail_w)
        def _():
            pltpu.make_async_copy(
                tout[0].at[pl.ds(0, 128 * D)],
                out_hbm.at[pl.ds(0, 128 * D)], osem[0]).wait()

        @pl.when(jnp.logical_not(tail_w))
        def _():
            wait_out(0)

        wait_out(1)

    return k


def _make_lookup(B0, B1):
    rows_per_w = B0 // NW
    n_chunks = rows_per_w // CHUNK_ROWS
    chunk = CHUNK_ROWS * B1
    assert n_chunks * CHUNK_ROWS == rows_per_w
    mesh = plsc.VectorSubcoreMesh(core_axis_name="c", subcore_axis_name="s")

    B1P = 32

    @functools.partial(
        pl.kernel,
        mesh=mesh,
        out_type=jax.ShapeDtypeStruct((B0, B1, D), jnp.float32),
        scratch_types=[pltpu.VMEM((CHUNK_ROWS, B1P), jnp.int32)] * NBUF
        + [pltpu.VMEM((chunk,), jnp.int32)] * NBUF
        + [pltpu.VMEM((chunk, D), jnp.float32)] * NBUF
        + [pltpu.SemaphoreType.DMA] * (2 * NBUF),
        compiler_params=pltpu.CompilerParams(
            use_tc_tiling_on_sc=False, needs_layout_passes=False),
    )
    def k(idx_hbm, table_hbm, out_hbm, *scratch):
        idx2_v = scratch[:NBUF]
        idxf_v = scratch[NBUF:2 * NBUF]
        rows_v = scratch[2 * NBUF:3 * NBUF]
        gsem = scratch[3 * NBUF:4 * NBUF]
        osem = scratch[4 * NBUF:]
        wid = lax.axis_index("s") * 2 + lax.axis_index("c")
        rbase = wid * rows_per_w
        lanes = lax.iota(jnp.int32, LANES)

        gathers = [None] * n_chunks
        stores = [None] * n_chunks

        def issue(i):
            b = i % NBUF
            if i >= NBUF:
                stores[i - NBUF].wait()
            r0 = rbase + i * CHUNK_ROWS
            pltpu.sync_copy(idx_hbm.at[pl.ds(r0, CHUNK_ROWS)], idx2_v[b])
            flat = idxf_v[b]
            idx2 = idx2_v[b]

            def fl(r, carry):
                base = r * B1 + lanes
                va = idx2[r, pl.ds(0, LANES)]
                vb = idx2[r, pl.ds(B1 - LANES, LANES)]
                plsc.store_scatter(flat, [base], va)
                plsc.store_scatter(flat, [base + (B1 - LANES)], vb)
                return carry

            lax.fori_loop(0, CHUNK_ROWS, fl, 0)
            gathers[i] = pltpu.async_copy(table_hbm.at[flat], rows_v[b],
                                          gsem[b])

        def drain(i):
            b = i % NBUF
            gathers[i].wait()
            r0 = rbase + i * CHUNK_ROWS

            def st(r, carry):
                pltpu.async_copy(rows_v[b].at[pl.ds(r * B1, B1)],
                                 out_hbm.at[r0 + r], osem[b])
                return carry

            lax.fori_loop(0, CHUNK_ROWS, st, 0)
            stores[i] = pltpu.make_async_copy(
                table_hbm.at[pl.ds(0, CHUNK_ROWS * B1)], rows_v[b], osem[b])

        for i in range(min(NBUF - 1, n_chunks)):
            issue(i)
        for i in range(n_chunks):
            if i + NBUF - 1 < n_chunks:
                issue(i + NBUF - 1)
            drain(i)
        for i in range(max(0, n_chunks - NBUF), n_chunks):
            stores[i].wait()

    return k


def kernel(token_ids, vocab):
    B0, B1 = token_ids.shape
    V = vocab.shape[0]
    tp = jnp.pad(token_ids.astype(jnp.int32), ((0, 0), (0, 32 - B1)))
    vt = vocab.T
    table_lin = _make_transpose(V)(vt, lax.slice(vt, (0, V - 128),
                                                 (vt.shape[0], V))).reshape(V, D)
    return _make_lookup(B0, B1)(tp, table_lin)


# speedup vs baseline: 1.6306x; 1.6306x over previous
import functools

import jax
import jax.numpy as jnp
from jax import lax
from jax.experimental import pallas as pl
from jax.experimental.pallas import tpu as pltpu
from jax.experimental.pallas import tpu_sc as plsc

D = 32
NW = 32
CHUNK_ROWS = 32
NBUF = 3
LANES = 16
W1 = 512


def _make_transpose(V):
    n_full = V // W1
    n_iter = -(-n_full // NW)
    n_pair = -(-n_iter // 2)
    mesh = plsc.VectorSubcoreMesh(core_axis_name="c", subcore_axis_name="s")

    @functools.partial(
        pl.kernel,
        mesh=mesh,
        out_type=jax.ShapeDtypeStruct((V * D,), jnp.float32),
        scratch_types=[pltpu.VMEM((D, W1), jnp.float32)] * 2
        + [pltpu.VMEM((W1 * D,), jnp.float32)] * 2
        + [pltpu.SemaphoreType.DMA] * 4,
        compiler_params=pltpu.CompilerParams(needs_layout_passes=False),
    )
    def k(tt_hbm, tl_hbm, out_hbm, *scr):
        tin = scr[0:2]
        tout = scr[2:4]
        isem = scr[4:6]
        osem = scr[6:8]
        wid = lax.axis_index("s") * 2 + lax.axis_index("c")
        lane32 = lax.iota(jnp.int32, LANES) * D

        def blk_of(kk):
            return wid + NW * kk

        def start_in(blk, s):
            c0 = blk * W1
            pltpu.async_copy(tt_hbm.at[:, pl.ds(c0, W1)], tin[s], isem[s])

        def wait_in(s):
            pltpu.make_async_copy(tt_hbm.at[:, pl.ds(0, W1)],
                                  tin[s], isem[s]).wait()

        def wait_out(s):
            pltpu.make_async_copy(
                tout[s], out_hbm.at[pl.ds(0, W1 * D)], osem[s]).wait()

        lanes = lax.iota(jnp.int32, LANES)
        skews = [(lanes + sk) & (LANES - 1) for sk in range(LANES)]
        dsts = [lane32 + skews[sk] for sk in range(LANES)]

        def transpose_into(tin_b, tout_b, ngroups):
            def body(g, carry):
                j0 = g * LANES
                colv = j0 + lanes
                for dd0 in (0, LANES):
                    for sk in range(LANES):
                        v = plsc.load_gather(tin_b, [dd0 + skews[sk], colv])
                        plsc.store_scatter(
                            tout_b, [(j0 * D + dd0) + dsts[sk]], v)
                return carry

            lax.fori_loop(0, ngroups, body, 0)

        @pl.when(blk_of(0) < n_full)
        def _():
            start_in(blk_of(0), 0)

        def pair(t, carry):
            for par in range(2):
                kk = 2 * t + par
                nkk = kk + 1

                @pl.when(blk_of(nkk) < n_full)
                def _():
                    start_in(blk_of(nkk), 1 - par)

                @pl.when(blk_of(kk) < n_full)
                def _():
                    wait_in(par)

                    @pl.when(t >= 1)
                    def _():
                        wait_out(par)

                    transpose_into(tin[par], tout[par], W1 // LANES)
                    pltpu.async_copy(
                        tout[par],
                        out_hbm.at[pl.ds(blk_of(kk) * (W1 * D), W1 * D)],
                        osem[par])
            return carry

        lax.fori_loop(0, n_pair, pair, 0)

        @pl.when((wid == NW - 1) & (V % W1 != 0))
        def _():
            c0 = V - 128
            wait_out(0)
            pltpu.async_copy(tl_hbm, tin[0].at[:, pl.ds(0, 128)], isem[0])
            pltpu.make_async_copy(
                tl_hbm, tin[0].at[:, pl.ds(0, 128)], isem[0]).wait()
            transpose_into(tin[0], tout[0], 128 // LANES)
            pltpu.async_copy(tout[0].at[pl.ds(0, 128 * D)],
                             out_hbm.at[pl.ds(c0 * D, 128 * D)], osem[0])

        tail_w = (wid == NW - 1) & (V % W1 != 0)

        @pl.when(tail_w)
        def _():
            pltpu.make_async_copy(
                tout[0].at[pl.ds(0, 128 * D)],
                out_hbm.at[pl.ds(0, 128 * D)], osem[0]).wait()

        @pl.when(jnp.logical_not(tail_w))
        def _():
            wait_out(0)

        wait_out(1)

    return k


def _make_lookup(B0, B1):
    rows_per_w = B0 // NW
    n_chunks = rows_per_w // CHUNK_ROWS
    chunk = CHUNK_ROWS * B1
    assert n_chunks * CHUNK_ROWS == rows_per_w
    mesh = plsc.VectorSubcoreMesh(core_axis_name="c", subcore_axis_name="s")

    B1P = 32

    @functools.partial(
        pl.kernel,
        mesh=mesh,
        out_type=jax.ShapeDtypeStruct((B0, B1, D), jnp.float32),
        scratch_types=[pltpu.VMEM((CHUNK_ROWS, B1P), jnp.int32)] * NBUF
        + [pltpu.VMEM((chunk,), jnp.int32)] * NBUF
        + [pltpu.VMEM((chunk, D), jnp.float32)] * NBUF
        + [pltpu.SemaphoreType.DMA] * (2 * NBUF),
        compiler_params=pltpu.CompilerParams(
            use_tc_tiling_on_sc=False, needs_layout_passes=False),
    )
    def k(idx_hbm, table_hbm, out_hbm, *scratch):
        idx2_v = scratch[:NBUF]
        idxf_v = scratch[NBUF:2 * NBUF]
        rows_v = scratch[2 * NBUF:3 * NBUF]
        gsem = scratch[3 * NBUF:4 * NBUF]
        osem = scratch[4 * NBUF:]
        wid = lax.axis_index("s") * 2 + lax.axis_index("c")
        rbase = wid * rows_per_w
        lanes = lax.iota(jnp.int32, LANES)

        gathers = [None] * n_chunks
        stores = [None] * n_chunks

        def issue(i):
            b = i % NBUF
            if i >= NBUF:
                stores[i - NBUF].wait()
            r0 = rbase + i * CHUNK_ROWS
            pltpu.sync_copy(idx_hbm.at[pl.ds(r0, CHUNK_ROWS)], idx2_v[b])
            flat = idxf_v[b]
            idx2 = idx2_v[b]

            def fl(r, carry):
                base = r * B1 + lanes
                va = idx2[r, pl.ds(0, LANES)]
                vb = idx2[r, pl.ds(B1 - LANES, LANES)]
                plsc.store_scatter(flat, [base], va)
                plsc.store_scatter(flat, [base + (B1 - LANES)], vb)
                return carry

            lax.fori_loop(0, CHUNK_ROWS, fl, 0)
            gathers[i] = pltpu.async_copy(table_hbm.at[flat], rows_v[b],
                                          gsem[b])

        def drain(i):
            b = i % NBUF
            gathers[i].wait()
            r0 = rbase + i * CHUNK_ROWS

            def st(r, carry):
                pltpu.async_copy(rows_v[b].at[pl.ds(r * B1, B1)],
                                 out_hbm.at[r0 + r], osem[b])
                return carry

            lax.fori_loop(0, CHUNK_ROWS, st, 0)
            stores[i] = pltpu.make_async_copy(
                table_hbm.at[pl.ds(0, CHUNK_ROWS * B1)], rows_v[b], osem[b])

        for i in range(min(NBUF - 1, n_chunks)):
            issue(i)
        for i in range(n_chunks):
            if i + NBUF - 1 < n_chunks:
                issue(i + NBUF - 1)
            drain(i)
        for i in range(max(0, n_chunks - NBUF), n_chunks):
            stores[i].wait()

    return k


def kernel(token_ids, vocab):
    B0, B1 = token_ids.shape
    V = vocab.shape[0]
    tp = jnp.pad(token_ids.astype(jnp.int32), ((0, 0), (0, 32 - B1)))
    vt = vocab.T
    table_lin = _make_transpose(V)(vt, lax.slice(vt, (0, V - 128),
                                                 (vt.shape[0], V))).reshape(V, D)
    return _make_lookup(B0, B1)(tp, table_lin)


# speedup vs baseline: 1.9822x; 1.2156x over previous
import functools

import jax
import jax.numpy as jnp
from jax import lax
from jax.experimental import pallas as pl
from jax.experimental.pallas import tpu as pltpu
from jax.experimental.pallas import tpu_sc as plsc

D = 32
NW = 32
CHUNK_ROWS = 32
NBUF = 3
LANES = 16
W1 = 512


def _make_transpose(V):
    n_full = V // W1
    n_iter = -(-n_full // NW)
    n_pair = -(-n_iter // 2)
    mesh = plsc.VectorSubcoreMesh(core_axis_name="c", subcore_axis_name="s")

    @functools.partial(
        pl.kernel,
        mesh=mesh,
        out_type=jax.ShapeDtypeStruct((V * D,), jnp.float32),
        scratch_types=[pltpu.VMEM((D, W1), jnp.float32)] * 2
        + [pltpu.VMEM((W1 * D,), jnp.float32)] * 2
        + [pltpu.SemaphoreType.DMA] * 4,
        compiler_params=pltpu.CompilerParams(needs_layout_passes=False),
    )
    def k(tt_hbm, tl_hbm, out_hbm, *scr):
        tin = scr[0:2]
        tout = scr[2:4]
        isem = scr[4:6]
        osem = scr[6:8]
        wid = lax.axis_index("s") * 2 + lax.axis_index("c")
        lane32 = lax.iota(jnp.int32, LANES) * D

        def blk_of(kk):
            return wid + NW * kk

        def start_in(blk, s):
            c0 = blk * W1
            pltpu.async_copy(tt_hbm.at[:, pl.ds(c0, W1)], tin[s], isem[s])

        def wait_in(s):
            pltpu.make_async_copy(tt_hbm.at[:, pl.ds(0, W1)],
                                  tin[s], isem[s]).wait()

        def wait_out(s):
            pltpu.make_async_copy(
                tout[s], out_hbm.at[pl.ds(0, W1 * D)], osem[s]).wait()

        lanes = lax.iota(jnp.int32, LANES)
        skews = [(lanes + sk) & (LANES - 1) for sk in range(LANES)]
        dsts = [lane32 + skews[sk] for sk in range(LANES)]

        def transpose_into(tin_b, tout_b, ngroups):
            def body(g, carry):
                j0 = g * LANES
                colv = j0 + lanes
                for dd0 in (0, LANES):
                    for sk in range(LANES):
                        v = plsc.load_gather(tin_b, [dd0 + skews[sk], colv])
                        plsc.store_scatter(
                            tout_b, [(j0 * D + dd0) + dsts[sk]], v)
                return carry

            lax.fori_loop(0, ngroups, body, 0)

        @pl.when(blk_of(0) < n_full)
        def _():
            start_in(blk_of(0), 0)

        def pair(t, carry):
            for par in range(2):
                kk = 2 * t + par
                nkk = kk + 1

                @pl.when(blk_of(nkk) < n_full)
                def _():
                    start_in(blk_of(nkk), 1 - par)

                @pl.when(blk_of(kk) < n_full)
                def _():
                    wait_in(par)

                    @pl.when(t >= 1)
                    def _():
                        wait_out(par)

                    transpose_into(tin[par], tout[par], W1 // LANES)
                    pltpu.async_copy(
                        tout[par],
                        out_hbm.at[pl.ds(blk_of(kk) * (W1 * D), W1 * D)],
                        osem[par])
            return carry

        lax.fori_loop(0, n_pair, pair, 0)

        @pl.when((wid == NW - 1) & (V % W1 != 0))
        def _():
            c0 = V - 128
            wait_out(0)
            pltpu.async_copy(tl_hbm, tin[0].at[:, pl.ds(0, 128)], isem[0])
            pltpu.make_async_copy(
                tl_hbm, tin[0].at[:, pl.ds(0, 128)], isem[0]).wait()
            transpose_into(tin[0], tout[0], 128 // LANES)
            pltpu.async_copy(tout[0].at[pl.ds(0, 128 * D)],
                             out_hbm.at[pl.ds(c0 * D, 128 * D)], osem[0])

        tail_w = (wid == NW - 1) & (V % W1 != 0)

        @pl.when(tail_w)
        def _():
            pltpu.make_async_copy(
                tout[0].at[pl.ds(0, 128 * D)],
                out_hbm.at[pl.ds(0, 128 * D)], osem[0]).wait()

        @pl.when(jnp.logical_not(tail_w))
        def _():
            wait_out(0)

        wait_out(1)

    return k


def _make_out_transpose(B0, B1):
    BD = B1 * D
    TB = 128
    blocks_per_w = B0 // TB // NW
    mesh = plsc.VectorSubcoreMesh(core_axis_name="c", subcore_axis_name="s")

    @functools.partial(
        pl.kernel,
        mesh=mesh,
        out_type=jax.ShapeDtypeStruct((B1, D, B0), jnp.float32),
        scratch_types=[pltpu.VMEM((TB * BD,), jnp.float32)]
        + [pltpu.VMEM((D, TB), jnp.float32)] * 2
        + [pltpu.SemaphoreType.DMA] * 3,
        compiler_params=pltpu.CompilerParams(needs_layout_passes=False),
    )
    def k(flat_hbm, out_hbm, *scr):
        fin = scr[0]
        stage = scr[1:3]
        insem = scr[3]
        osem = scr[4:6]
        wid = lax.axis_index("s") * 2 + lax.axis_index("c")
        lanes = lax.iota(jnp.int32, LANES)
        skews = [(lanes + sk) & (LANES - 1) for sk in range(LANES)]
        src_c = [lanes * BD + skews[sk] for sk in range(LANES)]

        def transpose_plane(b1, par):
            def body(g, carry):
                j0 = g * LANES
                for dd0 in (0, LANES):
                    sbase = j0 * BD + b1 * D + dd0
                    for sk in range(LANES):
                        v = plsc.load_gather(fin, [sbase + src_c[sk]])
                        plsc.store_scatter(stage[par],
                                           [dd0 + skews[sk], j0 + lanes], v)
                return carry

            lax.fori_loop(0, TB // LANES, body, 0)

        def out_start(b1, par, b0):
            pltpu.async_copy(stage[par],
                             out_hbm.at[b1, :, pl.ds(b0, TB)], osem[par])

        def out_wait(par):
            pltpu.make_async_copy(stage[par],
                                  out_hbm.at[0, :, pl.ds(0, TB)],
                                  osem[par]).wait()

        for kb in range(blocks_per_w):
            bb = wid * blocks_per_w + kb
            b0 = bb * TB
            pltpu.sync_copy(flat_hbm.at[pl.ds(bb * (TB * BD), TB * BD)], fin)
            for par in range(2):
                if kb > 0:
                    out_wait(par)
                transpose_plane(par, par)
                out_start(par, par, b0)

            def pair(p, carry):
                for par in range(2):
                    b1 = 2 * p + par
                    out_wait(par)
                    transpose_plane(b1, par)
                    out_start(b1, par, b0)
                return carry

            lax.fori_loop(1, B1 // 2, pair, 0)

        out_wait(0)
        out_wait(1)

    return k


def _make_lookup(B0, B1):
    rows_per_w = B0 // NW
    n_chunks = rows_per_w // CHUNK_ROWS
    chunk = CHUNK_ROWS * B1
    assert n_chunks * CHUNK_ROWS == rows_per_w
    mesh = plsc.VectorSubcoreMesh(core_axis_name="c", subcore_axis_name="s")

    B1P = 32

    @functools.partial(
        pl.kernel,
        mesh=mesh,
        out_type=jax.ShapeDtypeStruct((B0, B1, D), jnp.float32),
        scratch_types=[pltpu.VMEM((CHUNK_ROWS, B1P), jnp.int32)] * NBUF
        + [pltpu.VMEM((chunk,), jnp.int32)] * NBUF
        + [pltpu.VMEM((chunk, D), jnp.float32)] * NBUF
        + [pltpu.SemaphoreType.DMA] * (2 * NBUF),
        compiler_params=pltpu.CompilerParams(
            use_tc_tiling_on_sc=False, needs_layout_passes=False),
    )
    def k(idx_hbm, table_hbm, out_hbm, *scratch):
        idx2_v = scratch[:NBUF]
        idxf_v = scratch[NBUF:2 * NBUF]
        rows_v = scratch[2 * NBUF:3 * NBUF]
        gsem = scratch[3 * NBUF:4 * NBUF]
        osem = scratch[4 * NBUF:]
        wid = lax.axis_index("s") * 2 + lax.axis_index("c")
        rbase = wid * rows_per_w
        lanes = lax.iota(jnp.int32, LANES)

        gathers = [None] * n_chunks
        stores = [None] * n_chunks

        def issue(i):
            b = i % NBUF
            if i >= NBUF:
                stores[i - NBUF].wait()
            r0 = rbase + i * CHUNK_ROWS
            pltpu.sync_copy(idx_hbm.at[pl.ds(r0, CHUNK_ROWS)], idx2_v[b])
            flat = idxf_v[b]
            idx2 = idx2_v[b]

            def fl(r, carry):
                base = r * B1 + lanes
                va = idx2[r, pl.ds(0, LANES)]
                vb = idx2[r, pl.ds(B1 - LANES, LANES)]
                plsc.store_scatter(flat, [base], va)
                plsc.store_scatter(flat, [base + (B1 - LANES)], vb)
                return carry

            lax.fori_loop(0, CHUNK_ROWS, fl, 0)
            gathers[i] = pltpu.async_copy(table_hbm.at[flat], rows_v[b],
                                          gsem[b])

        def drain(i):
            b = i % NBUF
            gathers[i].wait()
            r0 = rbase + i * CHUNK_ROWS

            def st(r, carry):
                pltpu.async_copy(rows_v[b].at[pl.ds(r * B1, B1)],
                                 out_hbm.at[r0 + r], osem[b])
                return carry

            lax.fori_loop(0, CHUNK_ROWS, st, 0)
            stores[i] = pltpu.make_async_copy(
                table_hbm.at[pl.ds(0, CHUNK_ROWS * B1)], rows_v[b], osem[b])

        for i in range(min(NBUF - 1, n_chunks)):
            issue(i)
        for i in range(n_chunks):
            if i + NBUF - 1 < n_chunks:
                issue(i + NBUF - 1)
            drain(i)
        for i in range(max(0, n_chunks - NBUF), n_chunks):
            stores[i].wait()

    return k


def kernel(token_ids, vocab):
    B0, B1 = token_ids.shape
    V = vocab.shape[0]
    tp = jnp.pad(token_ids.astype(jnp.int32), ((0, 0), (0, 32 - B1)))
    vt = vocab.T
    table_lin = _make_transpose(V)(vt, lax.slice(vt, (0, V - 128),
                                                 (vt.shape[0], V))).reshape(V, D)
    out2 = _make_lookup(B0, B1)(tp, table_lin)
    out3 = _make_out_transpose(B0, B1)(out2.reshape(B0 * B1 * D))
    return jnp.transpose(out3, (2, 0, 1))
